# baseline (speedup 1.0000x reference)
import functools

import jax
import jax.numpy as jnp
from jax import lax
from jax.experimental import pallas as pl
from jax.experimental.pallas import tpu as pltpu
from jax.experimental.pallas import tpu_sc as plsc

_T, _C, _H, _W = 100, 3, 1024, 1024
_HW = _H * _W
_N = 2097152

_NC, _NS, _L = 2, 16, 16
_NW = _NC * _NS
_PPW = _N // _NW
_K = 2048
_G = _PPW // _K
_KI = 3 * _K


def _quantize(x, scale, hi):
    xf = x * scale
    yi = (xf + 0.5).astype(jnp.int32)
    is_half = (yi.astype(jnp.float32) - xf) == 0.5
    yi = yi - jnp.where(is_half, yi & 1, 0)
    return jnp.clip(yi, 0, hi)


def _sc_gather(t, h, w, weight_flat):
    mesh = plsc.VectorSubcoreMesh(core_axis_name="c", subcore_axis_name="s")

    @functools.partial(
        pl.kernel,
        out_type=jax.ShapeDtypeStruct((_N * _C,), jnp.float32),
        mesh=mesh,
        compiler_params=pltpu.CompilerParams(needs_layout_passes=False),
        scratch_types=[
            pltpu.VMEM((_K,), jnp.float32),
            pltpu.VMEM((_K,), jnp.float32),
            pltpu.VMEM((_K,), jnp.float32),
            pltpu.VMEM((_KI,), jnp.int32),
            pltpu.VMEM((_KI,), jnp.float32),
            pltpu.SemaphoreType.DMA,
        ],
    )
    def k(t_hbm, h_hbm, w_hbm, wt_hbm, out_hbm, tb, hb, wb, idx, vals, sem):
        wid = lax.axis_index("s") * _NC + lax.axis_index("c")
        lane3 = lax.iota(jnp.int32, _L) * 3

        def chunk(g, carry):
            off = wid * _PPW + g * _K
            pltpu.sync_copy(t_hbm.at[pl.ds(off, _K)], tb)
            pltpu.sync_copy(h_hbm.at[pl.ds(off, _K)], hb)
            pltpu.sync_copy(w_hbm.at[pl.ds(off, _K)], wb)

            def compute(j, c2):
                tv = tb[pl.ds(j * _L, _L)]
                hv = hb[pl.ds(j * _L, _L)]
                wv = wb[pl.ds(j * _L, _L)]
                mt = _quantize(tv, float(_T), _T - 1)
                mh = _quantize(hv, float(_H), _H - 1)
                mw = _quantize(wv, float(_W), _W - 1)
                idx0 = mt * (_C * _HW) + mh * _W + mw
                pos = j * (3 * _L) + lane3
                plsc.store_scatter(idx, [pos], idx0)
                plsc.store_scatter(idx, [pos + 1], idx0 + _HW)
                plsc.store_scatter(idx, [pos + 2], idx0 + 2 * _HW)
                return c2

            lax.fori_loop(0, _K // _L, compute, 0)
            pltpu.async_copy(wt_hbm.at[idx], vals, sem).wait()
            pltpu.sync_copy(vals, out_hbm.at[pl.ds(off * 3, _KI)])
            return carry

        lax.fori_loop(0, _G, chunk, 0)

    return k(t, h, w, weight_flat)


def kernel(t, h, w, weight):
    out_flat = _sc_gather(t, h, w, weight.reshape(-1))
    return out_flat.reshape(_N, _C)


# speedup vs baseline: 5.2545x; 5.2545x over previous
import functools

import jax
import jax.numpy as jnp
from jax import lax
from jax.experimental import pallas as pl
from jax.experimental.pallas import tpu as pltpu
from jax.experimental.pallas import tpu_sc as plsc

_T, _C, _H, _W = 100, 3, 1024, 1024
_HW = _H * _W
_N = 2097152

_NC, _NS, _L = 2, 16, 16
_NW = _NC * _NS
_PPW = _N // _NW
_K = 2048
_G = _PPW // _K
_KI = 3 * _K


def _quantize(x, scale, hi):
    xf = x * scale
    yi = (xf + 0.5).astype(jnp.int32)
    is_half = (yi.astype(jnp.float32) - xf) == 0.5
    yi = yi - jnp.where(is_half, yi & 1, 0)
    return jnp.clip(yi, 0, hi)


def _sc_gather(t, h, w, weight_flat):
    mesh = plsc.VectorSubcoreMesh(core_axis_name="c", subcore_axis_name="s")

    @functools.partial(
        pl.kernel,
        out_type=jax.ShapeDtypeStruct((4 * _N,), jnp.float32),
        mesh=mesh,
        compiler_params=pltpu.CompilerParams(needs_layout_passes=False),
        scratch_types=[
            pltpu.VMEM((_K,), jnp.float32),
            pltpu.VMEM((_K,), jnp.float32),
            pltpu.VMEM((_K,), jnp.float32),
            pltpu.VMEM((4 * _K,), jnp.int32),
            pltpu.VMEM((4 * _K,), jnp.float32),
            pltpu.SemaphoreType.DMA,
        ],
    )
    def k(t_hbm, h_hbm, w_hbm, wt_hbm, out_hbm, tb, hb, wb, idx, vals, sem):
        wid = lax.axis_index("s") * _NC + lax.axis_index("c")

        def chunk(g, carry):
            off = wid * _PPW + g * _K
            pltpu.sync_copy(t_hbm.at[pl.ds(off, _K)], tb)
            pltpu.sync_copy(h_hbm.at[pl.ds(off, _K)], hb)
            pltpu.sync_copy(w_hbm.at[pl.ds(off, _K)], wb)

            def compute(j, c2):
                tv = tb[pl.ds(j * _L, _L)]
                hv = hb[pl.ds(j * _L, _L)]
                wv = wb[pl.ds(j * _L, _L)]
                mt = _quantize(tv, float(_T), _T - 1)
                mh = _quantize(hv, float(_H), _H - 1)
                mw = _quantize(wv, float(_W), _W - 1)
                idx0 = (
                    mt * (_C * _HW)
                    + ((mh >> 3) << 13)
                    + ((mw >> 7) << 10)
                    + ((mh & 7) << 7)
                    + (mw & 127)
                )
                base = (j // 8) * 512 + (j % 8) * _L
                idx[pl.ds(base, _L)] = idx0
                idx[pl.ds(base + 128, _L)] = idx0 + _HW
                idx[pl.ds(base + 256, _L)] = idx0 + 2 * _HW
                idx[pl.ds(base + 384, _L)] = idx0
                return c2

            lax.fori_loop(0, _K // _L, compute, 0)
            pltpu.async_copy(wt_hbm.at[idx], vals, sem).wait()
            pltpu.sync_copy(vals, out_hbm.at[pl.ds(off * 4, 4 * _K)])
            return carry

        lax.fori_loop(0, _G, chunk, 0)

    return k(t, h, w, weight_flat)


def kernel(t, h, w, weight):
    wt_tiles = weight.reshape(_T * _C * _H // 8, 8, _W // 128, 128)
    wt_flat = wt_tiles.transpose(0, 2, 1, 3).reshape(-1)
    out_flat = _sc_gather(t, h, w, wt_flat)
    out3 = out_flat.reshape(_N // 128, 4, 128)
    return out3.transpose(0, 2, 1).reshape(_N, 4)[:, :_C]


# speedup vs baseline: 7.4726x; 1.4221x over previous
import functools

import jax
import jax.numpy as jnp
from jax import lax
from jax.experimental import pallas as pl
from jax.experimental.pallas import tpu as pltpu
from jax.experimental.pallas import tpu_sc as plsc

_T, _C, _H, _W = 100, 3, 1024, 1024
_HW = _H * _W
_N = 2097152

_NC, _NS, _L = 2, 16, 16
_NW = _NC * _NS
_PPW = _N // _NW
_K = 2048
_G = _PPW // _K
_KO = 4 * _K

def _quantize(x, scale, hi):
    y = x * scale + 2.0**23
    i = lax.bitcast_convert_type(y, jnp.int32) - 0x4B000000
    return jnp.minimum(i, hi)


def _sc_gather(t, h, w, weight_flat):
    mesh = plsc.VectorSubcoreMesh(core_axis_name="c", subcore_axis_name="s")

    @functools.partial(
        pl.kernel,
        out_type=jax.ShapeDtypeStruct((4 * _N,), jnp.float32),
        mesh=mesh,
        compiler_params=pltpu.CompilerParams(needs_layout_passes=False),
        scratch_types=[
            pltpu.VMEM((_K,), jnp.float32),
            pltpu.VMEM((_K,), jnp.float32),
            pltpu.VMEM((_K,), jnp.float32),
            pltpu.VMEM((_K,), jnp.float32),
            pltpu.VMEM((_K,), jnp.float32),
            pltpu.VMEM((_K,), jnp.float32),
            pltpu.VMEM((_KO,), jnp.int32),
            pltpu.VMEM((_KO,), jnp.int32),
            pltpu.VMEM((_KO,), jnp.float32),
            pltpu.VMEM((_KO,), jnp.float32),
            pltpu.SemaphoreType.DMA,
            pltpu.SemaphoreType.DMA,
        ],
    )
    def k(
        t_hbm, h_hbm, w_hbm, wt_hbm, out_hbm,
        tb0, hb0, wb0, tb1, hb1, wb1, ix0, ix1, va0, va1, sem_g, sem_o,
    ):
        wid = lax.axis_index("s") * _NC + lax.axis_index("c")

        def stage_in(g, tb, hb, wb):
            off = wid * _PPW + g * _K
            pltpu.sync_copy(t_hbm.at[pl.ds(off, _K)], tb)
            pltpu.sync_copy(h_hbm.at[pl.ds(off, _K)], hb)
            pltpu.sync_copy(w_hbm.at[pl.ds(off, _K)], wb)

        def compute(tb, hb, wb, ix):
            @plsc.parallel_loop(0, _K // _L, unroll=4)
            def _(j):
                tv = tb[pl.ds(j * _L, _L)]
                hv = hb[pl.ds(j * _L, _L)]
                wv = wb[pl.ds(j * _L, _L)]
                mt = _quantize(tv, float(_T), _T - 1)
                mh = _quantize(hv, float(_H), _H - 1)
                mw = _quantize(wv, float(_W), _W - 1)
                r = mt * (_C * _H) + mh
                idx0 = (
                    ((r >> 3) << 13)
                    | ((mw >> 7) << 10)
                    | ((r & 7) << 7)
                    | (mw & 127)
                )
                base = (j // 8) * 512 + (j % 8) * _L
                ix[pl.ds(base, _L)] = idx0
                ix[pl.ds(base + 128, _L)] = idx0 + _HW
                ix[pl.ds(base + 256, _L)] = idx0 + 2 * _HW
                ix[pl.ds(base + 384, _L)] = idx0

        def fire_gather(ix, va):
            pltpu.async_copy(wt_hbm.at[ix], va, sem_g)

        def wait_gather(va):
            pltpu.make_async_copy(wt_hbm.at[pl.ds(0, _KO)], va, sem_g).wait()

        def fire_out(g, va):
            off = wid * _PPW + g * _K
            pltpu.async_copy(va, out_hbm.at[pl.ds(4 * off, _KO)], sem_o)

        def wait_out(va):
            pltpu.make_async_copy(va, out_hbm.at[pl.ds(0, _KO)], sem_o).wait()

        stage_in(0, tb0, hb0, wb0)
        compute(tb0, hb0, wb0, ix0)
        fire_gather(ix0, va0)

        def outer(gg, carry):
            g0 = 2 * gg
            stage_in(g0 + 1, tb1, hb1, wb1)
            compute(tb1, hb1, wb1, ix1)
            wait_gather(va0)

            @pl.when(g0 >= 1)
            def _():
                wait_out(va1)

            fire_out(g0, va0)
            fire_gather(ix1, va1)

            @pl.when(g0 + 2 < _G)
            def _():
                stage_in(g0 + 2, tb0, hb0, wb0)
                compute(tb0, hb0, wb0, ix0)

            wait_gather(va1)
            wait_out(va0)
            fire_out(g0 + 1, va1)

            @pl.when(g0 + 2 < _G)
            def _():
                fire_gather(ix0, va0)

            return carry

        lax.fori_loop(0, _G // 2, outer, 0)
        wait_out(va1)

    return k(t, h, w, weight_flat)


def kernel(t, h, w, weight):
    wt_tiles = weight.reshape(_T * _C * _H // 8, 8, _W // 128, 128)
    wt_flat = wt_tiles.transpose(0, 2, 1, 3).reshape(-1)
    out_flat = _sc_gather(t, h, w, wt_flat)
    out3 = out_flat.reshape(_N // 128, 4, 128)
    return out3.transpose(0, 2, 1).reshape(_N, 4)[:, :_C]


# speedup vs baseline: 9.3841x; 1.2558x over previous
import functools

import jax
import jax.numpy as jnp
from jax import lax
from jax.experimental import pallas as pl
from jax.experimental.pallas import tpu as pltpu
from jax.experimental.pallas import tpu_sc as plsc

_T, _C, _H, _W = 100, 3, 1024, 1024
_HW = _H * _W
_N = 2097152

_NC, _NS, _L = 2, 16, 16
_NW = _NC * _NS
_PPW = _N // _NW
_K = 2048
_G = _PPW // _K
_KO = 4 * _K

def _quantize(x, scale, hi):
    y = x * scale + 2.0**23
    i = lax.bitcast_convert_type(y, jnp.int32) - 0x4B000000
    return jnp.minimum(i, hi)


def _sc_gather(t, h, w, weight_flat):
    mesh = plsc.VectorSubcoreMesh(core_axis_name="c", subcore_axis_name="s")

    @functools.partial(
        pl.kernel,
        out_type=jax.ShapeDtypeStruct((4 * _N,), jnp.float32),
        mesh=mesh,
        compiler_params=pltpu.CompilerParams(needs_layout_passes=False),
        scratch_types=[
            pltpu.VMEM((_K,), jnp.float32),
            pltpu.VMEM((_K,), jnp.float32),
            pltpu.VMEM((_K,), jnp.float32),
            pltpu.VMEM((_K,), jnp.float32),
            pltpu.VMEM((_K,), jnp.float32),
            pltpu.VMEM((_K,), jnp.float32),
            pltpu.VMEM((_C * _K,), jnp.int32),
            pltpu.VMEM((_C * _K,), jnp.int32),
            pltpu.VMEM((_KO,), jnp.float32),
            pltpu.VMEM((_KO,), jnp.float32),
            pltpu.SemaphoreType.DMA,
            pltpu.SemaphoreType.DMA,
        ],
    )
    def k(
        t_hbm, h_hbm, w_hbm, wt_hbm, out_hbm,
        tb0, hb0, wb0, tb1, hb1, wb1, ix0, ix1, va0, va1, sem_g, sem_o,
    ):
        wid = lax.axis_index("s") * _NC + lax.axis_index("c")

        def stage_in(g, tb, hb, wb):
            off = wid * _PPW + g * _K
            pltpu.sync_copy(t_hbm.at[pl.ds(off, _K)], tb)
            pltpu.sync_copy(h_hbm.at[pl.ds(off, _K)], hb)
            pltpu.sync_copy(w_hbm.at[pl.ds(off, _K)], wb)

        def compute(tb, hb, wb, ix):
            @plsc.parallel_loop(0, _K // _L, unroll=4)
            def _(j):
                tv = tb[pl.ds(j * _L, _L)]
                hv = hb[pl.ds(j * _L, _L)]
                wv = wb[pl.ds(j * _L, _L)]
                mt = _quantize(tv, float(_T), _T - 1)
                mh = _quantize(hv, float(_H), _H - 1)
                mw = _quantize(wv, float(_W), _W - 1)
                r = mt * (_C * _H) + mh
                idx0 = (
                    ((r >> 3) << 13)
                    | ((mw >> 7) << 10)
                    | ((r & 7) << 7)
                    | (mw & 127)
                )
                base = (j // 8) * 384 + (j % 8) * _L
                ix[pl.ds(base, _L)] = idx0
                ix[pl.ds(base + 128, _L)] = idx0 + _HW
                ix[pl.ds(base + 256, _L)] = idx0 + 2 * _HW

        def fire_gather(ix, va):
            def tile_body(tt, carry):
                for c in range(_C):
                    pltpu.async_copy(
                        wt_hbm.at[ix.at[pl.ds(tt * 384 + c * 128, 128)]],
                        va.at[pl.ds(tt * 512 + c * 128, 128)],
                        sem_g,
                    )
                return carry

            lax.fori_loop(0, _K // 128, tile_body, 0)

        def wait_gather(va):
            pltpu.make_async_copy(
                wt_hbm.at[pl.ds(0, _C * _K)],
                va.at[pl.ds(0, _C * _K)],
                sem_g,
            ).wait()

        def fire_out(g, va):
            off = wid * _PPW + g * _K
            pltpu.async_copy(va, out_hbm.at[pl.ds(4 * off, _KO)], sem_o)

        def wait_out(va):
            pltpu.make_async_copy(va, out_hbm.at[pl.ds(0, _KO)], sem_o).wait()

        stage_in(0, tb0, hb0, wb0)
        compute(tb0, hb0, wb0, ix0)
        fire_gather(ix0, va0)

        def outer(gg, carry):
            g0 = 2 * gg
            stage_in(g0 + 1, tb1, hb1, wb1)
            compute(tb1, hb1, wb1, ix1)
            wait_gather(va0)

            @pl.when(g0 >= 1)
            def _():
                wait_out(va1)

            fire_out(g0, va0)
            fire_gather(ix1, va1)

            @pl.when(g0 + 2 < _G)
            def _():
                stage_in(g0 + 2, tb0, hb0, wb0)
                compute(tb0, hb0, wb0, ix0)

            wait_gather(va1)
            wait_out(va0)
            fire_out(g0 + 1, va1)

            @pl.when(g0 + 2 < _G)
            def _():
                fire_gather(ix0, va0)

            return carry

        lax.fori_loop(0, _G // 2, outer, 0)
        wait_out(va1)

    return k(t, h, w, weight_flat)


def kernel(t, h, w, weight):
    wt_tiles = weight.reshape(_T * _C * _H // 8, 8, _W // 128, 128)
    wt_flat = wt_tiles.transpose(0, 2, 1, 3).reshape(-1)
    out_flat = _sc_gather(t, h, w, wt_flat)
    out3 = out_flat.reshape(_N // 128, 4, 128)
    return out3.transpose(0, 2, 1).reshape(_N, 4)[:, :_C]


# speedup vs baseline: 10.2574x; 1.0931x over previous
import functools

import jax
import jax.numpy as jnp
from jax import lax
from jax.experimental import pallas as pl
from jax.experimental.pallas import tpu as pltpu
from jax.experimental.pallas import tpu_sc as plsc

_T, _C, _H, _W = 100, 3, 1024, 1024
_HW = _H * _W
_N = 2097152

_NC, _NS, _L = 2, 16, 16
_NW = _NC * _NS
_PPW = _N // _NW
_K = 2048
_G = _PPW // _K
_KO = 4 * _K

def _quantize(x, scale, hi):
    y = x * scale + 2.0**23
    i = lax.bitcast_convert_type(y, jnp.int32) - 0x4B000000
    return jnp.minimum(i, hi)


def _sc_gather(t, h, w, weight_flat):
    mesh = plsc.VectorSubcoreMesh(core_axis_name="c", subcore_axis_name="s")

    @functools.partial(
        pl.kernel,
        out_type=jax.ShapeDtypeStruct((4 * _N,), jnp.float32),
        mesh=mesh,
        compiler_params=pltpu.CompilerParams(needs_layout_passes=False),
        scratch_types=[
            pltpu.VMEM((_K,), jnp.float32),
            pltpu.VMEM((_K,), jnp.float32),
            pltpu.VMEM((_K,), jnp.float32),
            pltpu.VMEM((_K,), jnp.float32),
            pltpu.VMEM((_K,), jnp.float32),
            pltpu.VMEM((_K,), jnp.float32),
            pltpu.VMEM((_C * _K,), jnp.int32),
            pltpu.VMEM((_C * _K,), jnp.int32),
            pltpu.VMEM((_KO,), jnp.float32),
            pltpu.VMEM((_KO,), jnp.float32),
            pltpu.SemaphoreType.DMA,
            pltpu.SemaphoreType.DMA,
            pltpu.SemaphoreType.DMA,
            pltpu.SemaphoreType.DMA,
            pltpu.SemaphoreType.DMA,
        ],
    )
    def k(
        t_hbm, h_hbm, w_hbm, wt_hbm, out_hbm,
        tb0, hb0, wb0, tb1, hb1, wb1, ix0, ix1, va0, va1,
        sg0, sg1, si0, si1, sem_o,
    ):
        wid = lax.axis_index("s") * _NC + lax.axis_index("c")

        def fire_in(g, tb, hb, wb, si):
            off = wid * _PPW + g * _K
            pltpu.async_copy(t_hbm.at[pl.ds(off, _K)], tb, si)
            pltpu.async_copy(h_hbm.at[pl.ds(off, _K)], hb, si)
            pltpu.async_copy(w_hbm.at[pl.ds(off, _K)], wb, si)

        def wait_in(tb, hb, wb, si):
            pltpu.make_async_copy(t_hbm.at[pl.ds(0, _K)], tb, si).wait()
            pltpu.make_async_copy(t_hbm.at[pl.ds(0, _K)], hb, si).wait()
            pltpu.make_async_copy(t_hbm.at[pl.ds(0, _K)], wb, si).wait()

        def compute(tb, hb, wb, ix):
            @plsc.parallel_loop(0, _K // _L, unroll=4)
            def _(j):
                tv = tb[pl.ds(j * _L, _L)]
                hv = hb[pl.ds(j * _L, _L)]
                wv = wb[pl.ds(j * _L, _L)]
                mt = _quantize(tv, float(_T), _T - 1)
                mh = _quantize(hv, float(_H), _H - 1)
                mw = _quantize(wv, float(_W), _W - 1)
                r = mt * (_C * _H) + mh
                idx0 = (
                    ((r >> 3) << 13)
                    | ((mw >> 7) << 10)
                    | ((r & 7) << 7)
                    | (mw & 127)
                )
                base = (j // 8) * 384 + (j % 8) * _L
                ix[pl.ds(base, _L)] = idx0
                ix[pl.ds(base + 128, _L)] = idx0 + _HW
                ix[pl.ds(base + 256, _L)] = idx0 + 2 * _HW

        def fire_gather(ix, va, sg):
            def tile_body(tt, carry):
                for c in range(_C):
                    pltpu.async_copy(
                        wt_hbm.at[ix.at[pl.ds(tt * 384 + c * 128, 128)]],
                        va.at[pl.ds(tt * 512 + c * 128, 128)],
                        sg,
                    )
                return carry

            lax.fori_loop(0, _K // 128, tile_body, 0)

        def wait_gather(va, sg):
            pltpu.make_async_copy(
                wt_hbm.at[pl.ds(0, _C * _K)],
                va.at[pl.ds(0, _C * _K)],
                sg,
            ).wait()

        def fire_out(g, va):
            off = wid * _PPW + g * _K
            pltpu.async_copy(va, out_hbm.at[pl.ds(4 * off, _KO)], sem_o)

        def wait_out(va):
            pltpu.make_async_copy(va, out_hbm.at[pl.ds(0, _KO)], sem_o).wait()

        fire_in(0, tb0, hb0, wb0, si0)
        fire_in(1, tb1, hb1, wb1, si1)
        wait_in(tb0, hb0, wb0, si0)
        compute(tb0, hb0, wb0, ix0)
        fire_gather(ix0, va0, sg0)

        def outer(gg, carry):
            g0 = 2 * gg
            @pl.when(g0 + 2 < _G)
            def _():
                fire_in(g0 + 2, tb0, hb0, wb0, si0)

            wait_in(tb1, hb1, wb1, si1)
            compute(tb1, hb1, wb1, ix1)

            @pl.when(g0 >= 1)
            def _():
                wait_out(va1)

            fire_gather(ix1, va1, sg1)
            wait_gather(va0, sg0)
            fire_out(g0, va0)

            @pl.when(g0 + 3 < _G)
            def _():
                fire_in(g0 + 3, tb1, hb1, wb1, si1)

            @pl.when(g0 + 2 < _G)
            def _():
                wait_in(tb0, hb0, wb0, si0)
                compute(tb0, hb0, wb0, ix0)

            wait_out(va0)

            @pl.when(g0 + 2 < _G)
            def _():
                fire_gather(ix0, va0, sg0)

            wait_gather(va1, sg1)
            fire_out(g0 + 1, va1)
            return carry

        lax.fori_loop(0, _G // 2, outer, 0)
        wait_out(va1)

    return k(t, h, w, weight_flat)


def kernel(t, h, w, weight):
    wt_tiles = weight.reshape(_T * _C * _H // 8, 8, _W // 128, 128)
    wt_flat = wt_tiles.transpose(0, 2, 1, 3).reshape(-1)
    out_flat = _sc_gather(t, h, w, wt_flat)
    out3 = out_flat.reshape(_N // 128, 4, 128)
    return out3.transpose(0, 2, 1).reshape(_N, 4)[:, :_C]
